# Initial kernel scaffold; baseline (speedup 1.0000x reference)
import functools

import jax
import jax.numpy as jnp
from jax import lax
from jax.experimental import pallas as pl
from jax.experimental.pallas import tpu as pltpu
from jax.experimental.pallas import tpu_sc as plsc

MAX_LENGTH = 256
OUT_DIM = 64

NUM_CORES = 2
NUM_SUBCORES = 16
NUM_WORKERS = NUM_CORES * NUM_SUBCORES

CHUNK = 1024


def _make_gather(batch: int):
  b_per_w = batch // NUM_WORKERS
  n_chunks = b_per_w // CHUNK
  mesh = plsc.VectorSubcoreMesh(core_axis_name="c", subcore_axis_name="s")

  @functools.partial(
      pl.kernel,
      mesh=mesh,
      out_type=jax.ShapeDtypeStruct((batch, OUT_DIM), jnp.float32),
      scratch_types=[
          pltpu.VMEM((CHUNK,), jnp.int32),
          pltpu.VMEM((CHUNK, OUT_DIM), jnp.float32),
          pltpu.SemaphoreType.DMA,
      ],
  )
  def gather_kernel(table_hbm, pos_hbm, out_hbm, idx_v, rows_v, sem):
    wid = lax.axis_index("s") * NUM_CORES + lax.axis_index("c")
    base = wid * b_per_w

    def body(i, carry):
      off = base + i * CHUNK
      pltpu.sync_copy(pos_hbm.at[pl.ds(off, CHUNK)], idx_v)
      pltpu.async_copy(table_hbm.at[idx_v], rows_v, sem).wait()
      pltpu.sync_copy(rows_v, out_hbm.at[pl.ds(off, CHUNK)])
      return carry

    lax.fori_loop(0, n_chunks, body, 0)

  return gather_kernel


def kernel(positions, table):
  batch, seq = positions.shape
  flat = positions.reshape(batch * seq)
  out = _make_gather(batch * seq)(table, flat)
  return out.reshape(batch, seq, OUT_DIM)


# speedup vs baseline: 3.0339x; 3.0339x over previous
import functools

import jax
import jax.numpy as jnp
from jax import lax
from jax.experimental import pallas as pl
from jax.experimental.pallas import tpu as pltpu
from jax.experimental.pallas import tpu_sc as plsc

MAX_LENGTH = 256
OUT_DIM = 64

NUM_CORES = 2
NUM_SUBCORES = 16
NUM_WORKERS = NUM_CORES * NUM_SUBCORES

CHUNK = 1024


def _make_gather(batch: int):
  b_per_w = batch // NUM_WORKERS
  n_chunks = b_per_w // CHUNK
  mesh = plsc.VectorSubcoreMesh(core_axis_name="c", subcore_axis_name="s")

  @functools.partial(
      pl.kernel,
      mesh=mesh,
      out_type=jax.ShapeDtypeStruct((batch, OUT_DIM), jnp.float32),
      scratch_types=[
          pltpu.VMEM((CHUNK,), jnp.int32),
          pltpu.VMEM((CHUNK, OUT_DIM), jnp.float32),
          pltpu.SemaphoreType.DMA,
      ],
      compiler_params=pltpu.CompilerParams(use_tc_tiling_on_sc=False),
  )
  def gather_kernel(table_hbm, pos_hbm, out_hbm, idx_v, rows_v, sem):
    wid = lax.axis_index("s") * NUM_CORES + lax.axis_index("c")
    base = wid * b_per_w

    def body(i, carry):
      off = base + i * CHUNK
      pltpu.sync_copy(pos_hbm.at[pl.ds(off, CHUNK)], idx_v)
      pltpu.async_copy(table_hbm.at[idx_v], rows_v, sem).wait()
      pltpu.sync_copy(rows_v, out_hbm.at[pl.ds(off, CHUNK)])
      return carry

    lax.fori_loop(0, n_chunks, body, 0)

  return gather_kernel


def kernel(positions, table):
  batch, seq = positions.shape
  flat = positions.reshape(batch * seq)
  out = _make_gather(batch * seq)(table, flat)
  return out.reshape(batch, seq, OUT_DIM)


# speedup vs baseline: 3.0375x; 1.0012x over previous
import functools

import jax
import jax.numpy as jnp
from jax import lax
from jax.experimental import pallas as pl
from jax.experimental.pallas import tpu as pltpu
from jax.experimental.pallas import tpu_sc as plsc

MAX_LENGTH = 256
OUT_DIM = 64

NUM_CORES = 2
NUM_SUBCORES = 16
NUM_WORKERS = NUM_CORES * NUM_SUBCORES

CHUNK = 800


def _make_gather(batch: int):
  b_per_w = batch // NUM_WORKERS
  n_chunks = b_per_w // CHUNK
  n_pairs = n_chunks // 2
  mesh = plsc.VectorSubcoreMesh(core_axis_name="c", subcore_axis_name="s")

  @functools.partial(
      pl.kernel,
      mesh=mesh,
      out_type=jax.ShapeDtypeStruct((batch, OUT_DIM), jnp.float32),
      scratch_types=[
          pltpu.VMEM((b_per_w,), jnp.int32),
          pltpu.VMEM((CHUNK, OUT_DIM), jnp.float32),
          pltpu.VMEM((CHUNK, OUT_DIM), jnp.float32),
          pltpu.SemaphoreType.DMA,
          pltpu.SemaphoreType.DMA,
          pltpu.SemaphoreType.DMA,
          pltpu.SemaphoreType.DMA,
      ],
      compiler_params=pltpu.CompilerParams(use_tc_tiling_on_sc=False),
  )
  def gather_kernel(table_hbm, pos_hbm, out_hbm, idx_all, rows0, rows1,
                    gsem0, gsem1, ssem0, ssem1):
    wid = lax.axis_index("s") * NUM_CORES + lax.axis_index("c")
    base = wid * b_per_w
    pltpu.sync_copy(pos_hbm.at[pl.ds(base, b_per_w)], idx_all)

    rows = (rows0, rows1)
    gsems = (gsem0, gsem1)
    ssems = (ssem0, ssem1)

    def gather_copy(i, b):
      return pltpu.make_async_copy(
          table_hbm.at[idx_all.at[pl.ds(i * CHUNK, CHUNK)]], rows[b],
          gsems[b])

    def store_copy(i, b):
      return pltpu.make_async_copy(
          rows[b], out_hbm.at[pl.ds(base + i * CHUNK, CHUNK)], ssems[b])

    gather_copy(0, 0).start()

    def body(j, carry):
      i0 = 2 * j
      i1 = 2 * j + 1

      @pl.when(j > 0)
      def _():
        store_copy(i0 - 1, 1).wait()

      gather_copy(i1, 1).start()
      gather_copy(i0, 0).wait()
      store_copy(i0, 0).start()

      @pl.when(j + 1 < n_pairs)
      def _():
        store_copy(i0, 0).wait()
        gather_copy(i0 + 2, 0).start()

      gather_copy(i1, 1).wait()
      store_copy(i1, 1).start()
      return carry

    lax.fori_loop(0, n_pairs, body, 0)

    store_copy(n_chunks - 2, 0).wait()
    store_copy(n_chunks - 1, 1).wait()

  return gather_kernel


def kernel(positions, table):
  batch, seq = positions.shape
  flat = positions.reshape(batch * seq)
  out = _make_gather(batch * seq)(table, flat)
  return out.reshape(batch, seq, OUT_DIM)
